# baseline (speedup 1.0000x reference)
import jax
import jax.numpy as jnp
from jax.experimental import pallas as pl
from jax.experimental.pallas import tpu as pltpu

_P = 4
_R = 256


def _mlp_block(x_ref, w1_ref, b1_ref, w2_ref, b2_ref, w3_ref, b3_ref,
               w4_ref, b4_ref, lo_ref, hi_ref, e_ref, o_ref):
    x32 = x_ref[...]
    xt_ = x32.T
    inb = ((xt_ >= lo_ref[...]) & (xt_ <= hi_ref[...])).astype(jnp.float32)
    pm3 = jnp.dot(e_ref[...], inb, preferred_element_type=jnp.float32)
    bf = jnp.bfloat16
    parts = []
    for q in range(8):
        x4 = xt_[16 * q:16 * q + 16, :].astype(bf)
        h = jnp.dot(w1_ref[...], x4, preferred_element_type=jnp.float32)
        h = jnp.maximum(h + b1_ref[...], 0.0).astype(bf)
        h = jnp.dot(w2_ref[...], h, preferred_element_type=jnp.float32)
        h = jnp.maximum(h + b2_ref[...], 0.0).astype(bf)
        h = jnp.dot(w3_ref[...], h, preferred_element_type=jnp.float32)
        h = jnp.maximum(h + b3_ref[...], 0.0).astype(bf)
        v = jnp.dot(w4_ref[...], h, preferred_element_type=jnp.float32)
        parts.append(v + b4_ref[...])
    vt = jnp.concatenate(parts, axis=0)
    res = jnp.where(pm3 == 3.0, vt, 0.0)
    o_ref[...] = res.T


def kernel(xt, bounds, W1, b1, W2, b2, W3, b3, W4, b4):
    n, d_in = xt.shape
    eye = jnp.eye(_P, dtype=jnp.float32)
    bf = jnp.bfloat16
    w1t = jnp.kron(eye, W1).T.astype(bf)
    w2t = jnp.kron(eye, W2).T.astype(bf)
    w3t = jnp.kron(eye, W3).T.astype(bf)
    w4t = jnp.kron(eye, W4).T.astype(bf)
    b1c = jnp.tile(b1, _P).reshape(-1, 1)
    b2c = jnp.tile(b2, _P).reshape(-1, 1)
    b3c = jnp.tile(b3, _P).reshape(-1, 1)
    b4c = jnp.tile(b4, _P).reshape(-1, 1)

    big = jnp.float32(3e38)
    lo = jnp.tile(jnp.concatenate([bounds[0], -big[None]]), 32).reshape(-1, 1)
    hi = jnp.tile(jnp.concatenate([bounds[1], big[None]]), 32).reshape(-1, 1)
    ri = jnp.arange(96)[:, None]
    ci = jnp.arange(128)[None, :]
    e = ((ci // 4 == ri // 3) & (ci % 4 < 3)).astype(jnp.float32)

    n32 = n // 32
    x32 = xt.reshape(n32, 128)
    grid = n32 // _R

    full = lambda r, c: pl.BlockSpec((r, c), lambda i: (0, 0))
    out = pl.pallas_call(
        _mlp_block,
        grid=(grid,),
        in_specs=[
            pl.BlockSpec((_R, 128), lambda i: (i, 0)),
            full(256, 16), full(256, 1),
            full(256, 256), full(256, 1),
            full(256, 256), full(256, 1),
            full(12, 256), full(12, 1),
            full(128, 1), full(128, 1),
            full(96, 128),
        ],
        out_specs=pl.BlockSpec((_R, 96), lambda i: (i, 0)),
        out_shape=jax.ShapeDtypeStruct((n32, 96), jnp.float32),
        compiler_params=pltpu.CompilerParams(
            dimension_semantics=("arbitrary",)),
    )(x32, w1t, b1c, w2t, b2c, w3t, b3c, w4t, b4c, lo, hi, e)
    return out.reshape(n, 3)


# speedup vs baseline: 1.3464x; 1.3464x over previous
import jax
import jax.numpy as jnp
from jax.experimental import pallas as pl
from jax.experimental.pallas import tpu as pltpu

_P = 4


def _mlp_block(x_ref, w1_ref, b1_ref, w2_ref, b2_ref, w3_ref, b3_ref,
               w4_ref, b4_ref, lo_ref, hi_ref, sel_ref, out_ref):
    xp = x_ref[...]
    h = jnp.dot(xp, w1_ref[...], preferred_element_type=jnp.float32)
    h = jnp.maximum(h + b1_ref[...], 0.0)
    h = jnp.dot(h, w2_ref[...], preferred_element_type=jnp.float32)
    h = jnp.maximum(h + b2_ref[...], 0.0)
    h = jnp.dot(h, w3_ref[...], preferred_element_type=jnp.float32)
    h = jnp.maximum(h + b3_ref[...], 0.0)
    v = jnp.dot(h, w4_ref[...], preferred_element_type=jnp.float32)
    v = v + b4_ref[...]
    inb = ((xp >= lo_ref[...]) & (xp <= hi_ref[...])).astype(jnp.float32)
    ind = jnp.dot(inb, sel_ref[...], preferred_element_type=jnp.float32)
    out_ref[...] = jnp.where(ind == 4.0, v, 0.0)


def kernel(xt, bounds, W1, b1, W2, b2, W3, b3, W4, b4):
    n, d_in = xt.shape
    d_h = W1.shape[1]
    d_out = W4.shape[1]
    eye = jnp.eye(_P, dtype=jnp.float32)
    w1p = jnp.kron(eye, W1)
    w2p = jnp.kron(eye, W2)
    w3p = jnp.kron(eye, W3)
    w4p = jnp.kron(eye, W4)
    b1p = jnp.tile(b1, _P).reshape(1, -1)
    b2p = jnp.tile(b2, _P).reshape(1, -1)
    b3p = jnp.tile(b3, _P).reshape(1, -1)
    b4p = jnp.tile(b4, _P).reshape(1, -1)
    big = jnp.float32(3e38)
    lo = jnp.tile(jnp.concatenate([bounds[0], -big[None]]), _P).reshape(1, -1)
    hi = jnp.tile(jnp.concatenate([bounds[1], big[None]]), _P).reshape(1, -1)
    li = jax.lax.broadcasted_iota(jnp.int32, (4 * _P, 3 * _P), 0) // 4
    lj = jax.lax.broadcasted_iota(jnp.int32, (4 * _P, 3 * _P), 1) // 3
    sel = (li == lj).astype(jnp.float32)

    bp = 2048
    npk = n // _P
    xp_all = xt.reshape(npk, _P * d_in)
    grid = npk // bp

    full = lambda r, c: pl.BlockSpec((r, c), lambda i: (0, 0))
    out = pl.pallas_call(
        _mlp_block,
        grid=(grid,),
        in_specs=[
            pl.BlockSpec((bp, _P * d_in), lambda i: (i, 0)),
            full(d_in * _P, d_h * _P),
            full(1, d_h * _P),
            full(d_h * _P, d_h * _P),
            full(1, d_h * _P),
            full(d_h * _P, d_h * _P),
            full(1, d_h * _P),
            full(d_h * _P, d_out * _P),
            full(1, d_out * _P),
            full(1, d_in * _P),
            full(1, d_in * _P),
            full(d_in * _P, d_out * _P),
        ],
        out_specs=pl.BlockSpec((bp, _P * d_out), lambda i: (i, 0)),
        out_shape=jax.ShapeDtypeStruct((npk, _P * d_out), jnp.float32),
        compiler_params=pltpu.CompilerParams(
            dimension_semantics=("arbitrary",)),
    )(xp_all, w1p, b1p, w2p, b2p, w3p, b3p, w4p, b4p, lo, hi, sel)
    return out.reshape(n, d_out)
